# Initial kernel scaffold; baseline (speedup 1.0000x reference)
import jax
import jax.numpy as jnp
from jax.experimental import pallas as pl


def kernel(h, edge_indexT, D):
    raise NotImplementedError("write your pallas kernel here")


# speedup vs baseline: 3.6105x; 3.6105x over previous
import functools

import jax
import jax.numpy as jnp
from jax import lax
from jax.experimental import pallas as pl
from jax.experimental.pallas import tpu as pltpu, tpu_sc as plsc

GAMMA = 0.1
NC = 2
NS = 16
NW = NC * NS
L = 16
CHUNK_E = 128
ROWCHUNK = 64


def _mesh():
    return plsc.VectorSubcoreMesh(core_axis_name="c", subcore_axis_name="s")


def _make_accum(n_pad, f, e_chunks_per_tile):
    ept = e_chunks_per_tile * CHUNK_E
    copy_steps = n_pad // NS // ROWCHUNK

    @functools.partial(
        pl.kernel,
        mesh=_mesh(),
        out_type=jax.ShapeDtypeStruct((NC, n_pad, f), jnp.float32),
        scratch_types=[
            pltpu.VMEM((CHUNK_E,), jnp.int32),
            pltpu.VMEM((CHUNK_E,), jnp.int32),
            pltpu.VMEM((CHUNK_E, f), jnp.float32),
            pltpu.VMEM((ROWCHUNK, f), jnp.float32),
            pltpu.VMEM_SHARED((n_pad, f), jnp.float32),
            pltpu.SemaphoreType.DMA,
        ],
    )
    def accum(h_hbm, src_hbm, dst_hbm, part_hbm, src_v, dst_v, rows_v,
              buf_v, agg_sh, sem):
        cid = lax.axis_index("c")
        sid = lax.axis_index("s")
        wid = cid * NS + sid

        def zbody(i, _):
            r = i // (f // L)
            cc = i % (f // L)
            buf_v[r, pl.ds(cc * L, L)] = jnp.zeros((L,), jnp.float32)
            return 0
        lax.fori_loop(0, ROWCHUNK * (f // L), zbody, 0)

        def zcopy(k, _):
            row0 = sid * (n_pad // NS) + k * ROWCHUNK
            pltpu.sync_copy(buf_v, agg_sh.at[pl.ds(row0, ROWCHUNK)])
            return 0
        lax.fori_loop(0, copy_steps, zcopy, 0)
        plsc.subcore_barrier()

        def ebody(j, _):
            base = wid * ept + j * CHUNK_E
            pltpu.sync_copy(src_hbm.at[pl.ds(base, CHUNK_E)], src_v)
            pltpu.sync_copy(dst_hbm.at[pl.ds(base, CHUNK_E)], dst_v)
            pltpu.async_copy(h_hbm.at[src_v], rows_v, sem).wait()
            pltpu.sync_copy(rows_v, agg_sh.at[dst_v], add=True)
            return 0
        lax.fori_loop(0, e_chunks_per_tile, ebody, 0)
        plsc.subcore_barrier()

        def ocopy(k, _):
            row0 = sid * (n_pad // NS) + k * ROWCHUNK
            pltpu.sync_copy(agg_sh.at[pl.ds(row0, ROWCHUNK)], buf_v)
            pltpu.sync_copy(buf_v, part_hbm.at[cid, pl.ds(row0, ROWCHUNK)])
            return 0
        lax.fori_loop(0, copy_steps, ocopy, 0)

    return accum


def _make_combine(n_pad, f, row_chunks_per_tile):
    @functools.partial(
        pl.kernel,
        mesh=_mesh(),
        out_type=jax.ShapeDtypeStruct((n_pad, f), jnp.float32),
        scratch_types=[
            pltpu.VMEM((ROWCHUNK, f), jnp.float32),
            pltpu.VMEM((ROWCHUNK, f), jnp.float32),
            pltpu.VMEM((ROWCHUNK, f), jnp.float32),
            pltpu.VMEM((ROWCHUNK,), jnp.float32),
        ],
    )
    def combine(h_hbm, part_hbm, d_hbm, out_hbm, hv, p0v, p1v, dv):
        cid = lax.axis_index("c")
        sid = lax.axis_index("s")
        wid = cid * NS + sid

        def cbody(k, _):
            row0 = (wid * row_chunks_per_tile + k) * ROWCHUNK
            pltpu.sync_copy(h_hbm.at[pl.ds(row0, ROWCHUNK)], hv)
            pltpu.sync_copy(part_hbm.at[0, pl.ds(row0, ROWCHUNK)], p0v)
            pltpu.sync_copy(part_hbm.at[1, pl.ds(row0, ROWCHUNK)], p1v)
            pltpu.sync_copy(d_hbm.at[pl.ds(row0, ROWCHUNK)], dv)

            def gbody(g, _):
                dvec = dv[pl.ds(g * L, L)]
                invv = 1.0 / (1.0 + GAMMA * dvec)
                for rr in range(L):
                    r = g * L + rr
                    inv_b = jnp.full((L,), invv[rr], jnp.float32)

                    def fbody(cc, _, r=r, inv_b=inv_b):
                        col = cc * L
                        agg = p0v[r, pl.ds(col, L)] + p1v[r, pl.ds(col, L)]
                        hv[r, pl.ds(col, L)] = (
                            hv[r, pl.ds(col, L)] + GAMMA * agg) * inv_b
                        return 0
                    lax.fori_loop(0, f // L, fbody, 0)
                return 0
            lax.fori_loop(0, ROWCHUNK // L, gbody, 0)
            pltpu.sync_copy(hv, out_hbm.at[pl.ds(row0, ROWCHUNK)])
            return 0
        lax.fori_loop(0, row_chunks_per_tile, cbody, 0)

    return combine


def kernel(h, edge_indexT, D):
    n, f = h.shape
    e = edge_indexT.shape[1]

    e_chunks_per_tile = -(-e // (NW * CHUNK_E))
    e_pad = NW * CHUNK_E * e_chunks_per_tile
    row_chunks_per_tile = -(-n // (NW * ROWCHUNK))
    n_pad = NW * ROWCHUNK * row_chunks_per_tile

    src = jnp.concatenate(
        [edge_indexT[0], jnp.zeros((e_pad - e,), jnp.int32)])
    dst = jnp.concatenate(
        [edge_indexT[1], jnp.full((e_pad - e,), n, jnp.int32)])
    h_pad = jnp.zeros((n_pad, f), jnp.float32).at[:n].set(h)
    d_pad = jnp.zeros((n_pad,), jnp.float32).at[:n].set(D)

    accum = _make_accum(n_pad, f, e_chunks_per_tile)
    combine = _make_combine(n_pad, f, row_chunks_per_tile)

    cur = h_pad
    for _ in range(2):
        part = accum(cur, src, dst)
        cur = combine(cur, part, d_pad)
    return cur[:n]
